# baseline (speedup 1.0000x reference)
import functools

import jax
import jax.numpy as jnp
from jax import lax
from jax.experimental import pallas as pl
from jax.experimental.pallas import tpu as pltpu
from jax.experimental.pallas import tpu_sc as plsc

NUM_CODES_K = 1024
DIM_K = 64
ROWS = 9216
ROW_BLOCK = 1152
NBLK = ROWS // ROW_BLOCK

NW = 32
B_PER_W = ROWS // NW
CHUNKS = 3
CHUNK = B_PER_W // CHUNKS


def _dist_argmin_body(xt_ref, cb_ref, idx_ref, acc_ref):
    xt = xt_ref[...]
    cb = cb_ref[...]
    mm = lax.dot_general(cb, xt, (((1,), (0,)), ((), ())),
                         preferred_element_type=jnp.float32)
    x2 = jnp.sum(xt * xt, axis=0, keepdims=True)
    c2 = jnp.sum(cb * cb, axis=1, keepdims=True)
    d = (x2 + c2) - 2.0 * mm
    m = jnp.min(d, axis=0, keepdims=True)
    iota = lax.broadcasted_iota(jnp.int32, d.shape, 0).astype(jnp.float32)
    idx = jnp.min(jnp.where(d == m, iota, float(NUM_CODES_K)), axis=0)
    idx_ref[0, 0, :] = idx.astype(jnp.int32)

    @pl.when(pl.program_id(0) == 0)
    def _():
        acc_ref[0, 0] = 0.0

    acc_ref[0, 0] += jnp.sum(m)


def _dist_argmin(xt, cb):
    return pl.pallas_call(
        _dist_argmin_body,
        grid=(NBLK,),
        in_specs=[
            pl.BlockSpec((DIM_K, ROW_BLOCK), lambda i: (0, i)),
            pl.BlockSpec((NUM_CODES_K, DIM_K), lambda i: (0, 0)),
        ],
        out_specs=[
            pl.BlockSpec((1, 1, ROW_BLOCK), lambda i: (i, 0, 0)),
            pl.BlockSpec(memory_space=pltpu.SMEM),
        ],
        out_shape=[
            jax.ShapeDtypeStruct((NBLK, 1, ROW_BLOCK), jnp.int32),
            jax.ShapeDtypeStruct((1, 1), jnp.float32),
        ],
    )(xt, cb)


@functools.cache
def _make_sc_gather():
    @functools.partial(
        pl.kernel,
        mesh=plsc.VectorSubcoreMesh(core_axis_name="c", subcore_axis_name="s"),
        out_type=[jax.ShapeDtypeStruct((NW, B_PER_W, DIM_K), jnp.float32),
                  jax.ShapeDtypeStruct((NW, B_PER_W, DIM_K), jnp.float32)],
        scratch_types=[
            pltpu.VMEM((CHUNKS, CHUNK), jnp.int32),
            pltpu.VMEM((B_PER_W, DIM_K), jnp.float32),
            pltpu.SemaphoreType.DMA,
        ],
        compiler_params=pltpu.CompilerParams(use_tc_tiling_on_sc=False),
    )
    def _sc_gather(cb_hbm, idx_hbm, out_hbm, out2_hbm, idx_v, rows_v, sem):
        wid = lax.axis_index("s") * 2 + lax.axis_index("c")
        pltpu.sync_copy(idx_hbm.at[wid], idx_v)
        copies = [
            pltpu.async_copy(cb_hbm.at[idx_v.at[k]],
                             rows_v.at[pl.ds(k * CHUNK, CHUNK)], sem)
            for k in range(CHUNKS)
        ]
        for cp in copies:
            cp.wait()
        pltpu.sync_copy(rows_v, out_hbm.at[wid])
        pltpu.sync_copy(rows_v, out2_hbm.at[wid])

    return _sc_gather


def kernel(inputs, codebook):
    xt = inputs.reshape(-1, DIM_K).T
    idx3, acc = _dist_argmin(xt, codebook)
    idx = idx3.reshape(NW, CHUNKS, CHUNK)
    q1, q2 = _make_sc_gather()(codebook, idx)
    quantized_out = q1.reshape(inputs.shape)
    quantized = q2.reshape(inputs.shape)
    loss = acc[0, 0] / float(ROWS * DIM_K)
    enc = idx3.reshape(inputs.shape[:-1])
    return (quantized_out, loss, loss, quantized, enc)


# speedup vs baseline: 1.0867x; 1.0867x over previous
import functools

import jax
import jax.numpy as jnp
from jax import lax
from jax.experimental import pallas as pl
from jax.experimental.pallas import tpu as pltpu
from jax.experimental.pallas import tpu_sc as plsc

NUM_CODES_K = 1024
DIM_K = 64
ROWS = 9216
ROW_BLOCK = 2304
NBLK = ROWS // ROW_BLOCK

NW = 32
B_PER_W = ROWS // NW
CHUNKS = 3
CHUNK = B_PER_W // CHUNKS


def _dist_argmin_body(xt_ref, cb_ref, idx_ref, acc_ref):
    xt = xt_ref[...]
    cb = cb_ref[...]
    mm = lax.dot_general(cb, xt, (((1,), (0,)), ((), ())),
                         preferred_element_type=jnp.float32)
    x2 = jnp.sum(xt * xt, axis=0, keepdims=True)
    c2 = jnp.sum(cb * cb, axis=1, keepdims=True)
    d = (x2 + c2) - 2.0 * mm
    m = jnp.min(d, axis=0, keepdims=True)
    iota = lax.broadcasted_iota(jnp.int32, d.shape, 0).astype(jnp.float32)
    idx = jnp.min(jnp.where(d == m, iota, float(NUM_CODES_K)), axis=0)
    idx_ref[0, 0, :] = idx.astype(jnp.int32)

    @pl.when(pl.program_id(0) == 0)
    def _():
        acc_ref[0, 0] = 0.0

    acc_ref[0, 0] += jnp.sum(m)


def _dist_argmin(xt, cb):
    return pl.pallas_call(
        _dist_argmin_body,
        grid=(NBLK,),
        in_specs=[
            pl.BlockSpec((DIM_K, ROW_BLOCK), lambda i: (0, i)),
            pl.BlockSpec((NUM_CODES_K, DIM_K), lambda i: (0, 0)),
        ],
        out_specs=[
            pl.BlockSpec((1, 1, ROW_BLOCK), lambda i: (i, 0, 0)),
            pl.BlockSpec(memory_space=pltpu.SMEM),
        ],
        out_shape=[
            jax.ShapeDtypeStruct((NBLK, 1, ROW_BLOCK), jnp.int32),
            jax.ShapeDtypeStruct((1, 1), jnp.float32),
        ],
    )(xt, cb)


@functools.cache
def _make_sc_gather():
    @functools.partial(
        pl.kernel,
        mesh=plsc.VectorSubcoreMesh(core_axis_name="c", subcore_axis_name="s"),
        out_type=jax.ShapeDtypeStruct((NW, B_PER_W, DIM_K), jnp.float32),
        scratch_types=[
            pltpu.VMEM((CHUNKS, CHUNK), jnp.int32),
            pltpu.VMEM((B_PER_W, DIM_K), jnp.float32),
            pltpu.SemaphoreType.DMA,
        ],
        compiler_params=pltpu.CompilerParams(use_tc_tiling_on_sc=False),
    )
    def _sc_gather(cb_hbm, idx_hbm, out_hbm, idx_v, rows_v, sem):
        wid = lax.axis_index("s") * 2 + lax.axis_index("c")
        pltpu.sync_copy(idx_hbm.at[wid], idx_v)
        copies = [
            pltpu.async_copy(cb_hbm.at[idx_v.at[k]],
                             rows_v.at[pl.ds(k * CHUNK, CHUNK)], sem)
            for k in range(CHUNKS)
        ]
        for cp in copies:
            cp.wait()
        pltpu.sync_copy(rows_v, out_hbm.at[wid])

    return _sc_gather


def kernel(inputs, codebook):
    xt = inputs.reshape(-1, DIM_K).T
    idx3, acc = _dist_argmin(xt, codebook)
    idx = idx3.reshape(NW, CHUNKS, CHUNK)
    quantized = _make_sc_gather()(codebook, idx).reshape(inputs.shape)
    loss = acc[0, 0] / float(ROWS * DIM_K)
    enc = idx3.reshape(inputs.shape[:-1])
    return (quantized, loss, loss, quantized, enc)


# speedup vs baseline: 1.0964x; 1.0089x over previous
import functools

import jax
import jax.numpy as jnp
from jax import lax
from jax.experimental import pallas as pl
from jax.experimental.pallas import tpu as pltpu
from jax.experimental.pallas import tpu_sc as plsc

NUM_CODES_K = 1024
DIM_K = 64
ROWS = 9216
ROW_BLOCK = 4608
NBLK = ROWS // ROW_BLOCK

NW = 32
B_PER_W = ROWS // NW
CHUNKS = 3
CHUNK = B_PER_W // CHUNKS


def _dist_argmin_body(xt_ref, cb_ref, idx_ref, acc_ref):
    xt = xt_ref[...]
    cb = cb_ref[...]
    mm = lax.dot_general(cb, xt, (((1,), (0,)), ((), ())),
                         preferred_element_type=jnp.float32)
    x2 = jnp.sum(xt * xt, axis=0, keepdims=True)
    c2 = jnp.sum(cb * cb, axis=1, keepdims=True)
    d = (x2 + c2) - 2.0 * mm
    m = jnp.min(d, axis=0, keepdims=True)
    iota = lax.broadcasted_iota(jnp.int32, d.shape, 0).astype(jnp.float32)
    idx = jnp.min(jnp.where(d == m, iota, float(NUM_CODES_K)), axis=0)
    idx_ref[0, 0, :] = idx.astype(jnp.int32)

    @pl.when(pl.program_id(0) == 0)
    def _():
        acc_ref[0, 0] = 0.0

    acc_ref[0, 0] += jnp.sum(m)


def _dist_argmin(xt, cb):
    return pl.pallas_call(
        _dist_argmin_body,
        grid=(NBLK,),
        in_specs=[
            pl.BlockSpec((DIM_K, ROW_BLOCK), lambda i: (0, i)),
            pl.BlockSpec((NUM_CODES_K, DIM_K), lambda i: (0, 0)),
        ],
        out_specs=[
            pl.BlockSpec((1, 1, ROW_BLOCK), lambda i: (i, 0, 0)),
            pl.BlockSpec(memory_space=pltpu.SMEM),
        ],
        out_shape=[
            jax.ShapeDtypeStruct((NBLK, 1, ROW_BLOCK), jnp.int32),
            jax.ShapeDtypeStruct((1, 1), jnp.float32),
        ],
    )(xt, cb)


@functools.cache
def _make_sc_gather():
    @functools.partial(
        pl.kernel,
        mesh=plsc.VectorSubcoreMesh(core_axis_name="c", subcore_axis_name="s"),
        out_type=jax.ShapeDtypeStruct((NW, B_PER_W, DIM_K), jnp.float32),
        scratch_types=[
            pltpu.VMEM((CHUNKS, CHUNK), jnp.int32),
            pltpu.VMEM((B_PER_W, DIM_K), jnp.float32),
            pltpu.SemaphoreType.DMA,
        ],
        compiler_params=pltpu.CompilerParams(use_tc_tiling_on_sc=False),
    )
    def _sc_gather(cb_hbm, idx_hbm, out_hbm, idx_v, rows_v, sem):
        wid = lax.axis_index("s") * 2 + lax.axis_index("c")
        pltpu.sync_copy(idx_hbm.at[wid], idx_v)
        copies = [
            pltpu.async_copy(cb_hbm.at[idx_v.at[k]],
                             rows_v.at[pl.ds(k * CHUNK, CHUNK)], sem)
            for k in range(CHUNKS)
        ]
        for cp in copies:
            cp.wait()
        pltpu.sync_copy(rows_v, out_hbm.at[wid])

    return _sc_gather


def kernel(inputs, codebook):
    xt = inputs.reshape(-1, DIM_K).T
    idx3, acc = _dist_argmin(xt, codebook)
    idx = idx3.reshape(NW, CHUNKS, CHUNK)
    quantized = _make_sc_gather()(codebook, idx).reshape(inputs.shape)
    loss = acc[0, 0] / float(ROWS * DIM_K)
    enc = idx3.reshape(inputs.shape[:-1])
    return (quantized, loss, loss, quantized, enc)
